# baseline (speedup 1.0000x reference)
import functools
import math

import numpy as _np

import jax
import jax.numpy as jnp
from jax import lax
from jax.experimental import pallas as pl
from jax.experimental.pallas import tpu as pltpu
from jax.experimental.pallas import tpu_sc as plsc

_CLASSES = 10
_SIZE = 64
_CMS = 8
_CTX = 128
_IN = 512
_B = 128
_ROWS = 2 ** _CMS
_LR = 0.01
_PRED_CLIP = 0.01
_WEIGHT_CLIP = 5.0
_CS = _CLASSES * _SIZE
_NDUMMY = 256
_NW = 32
_BLOCKS_PER_W = _CS // _NW


def _logit(x):
    return jnp.log(x) - jnp.log1p(-x)


def _index_body(maps_ref, bias_ref, ctxT_ref, gidg_ref, gids_ref):
    f32 = jnp.float32
    c = pl.program_id(0)
    s = pl.program_id(1)
    t = c * _SIZE + s
    d = jnp.dot(maps_ref[0, 0], ctxT_ref[...],
                preferred_element_type=f32)
    bits = (d > bias_ref[0]).astype(jnp.int32)
    pow2 = jnp.left_shift(
        1, lax.broadcasted_iota(jnp.int32, (_CMS, _B), 0))
    idx = jnp.sum(bits * pow2, axis=0, keepdims=True)
    idx_t = jnp.transpose(idx)
    eq = (idx_t == idx)
    sub_i = lax.broadcasted_iota(jnp.int32, (_B, _B), 0)
    lane_i = lax.broadcasted_iota(jnp.int32, (_B, _B), 1)
    loser = jnp.any(eq & (sub_i > lane_i), axis=0, keepdims=True)
    gidg = t * _ROWS + idx
    lane = lax.broadcasted_iota(jnp.int32, (1, _B), 1)
    dummy = _CS * _ROWS + ((t * _B + lane) % _NDUMMY)
    gidg_ref[0] = gidg
    gids_ref[0] = jnp.where(loser, dummy, gidg)


_LO = float(
    math.log(float(_np.float32(_PRED_CLIP)))
    - math.log1p(-float(_np.float32(_PRED_CLIP))))
_HI = float(
    math.log(float(_np.float32(1.0 - _PRED_CLIP)))
    - math.log1p(-float(_np.float32(1.0 - _PRED_CLIP))))


def _sc_body(w2_hbm, gidg_hbm, gids_hbm, lgt_hbm, tgt_hbm,
             out_hbm, olg_hbm,
             sel_v, lg_v, idxg_v, idxs_v, tgt_v, diff_v, olg_v, sem):
    f32 = jnp.float32
    lo = _np.float32(_LO)
    hi = _np.float32(_HI)
    wid = lax.axis_index("s") * 2 + lax.axis_index("c")
    iota16 = lax.broadcasted_iota(jnp.int32, (16,), 0)
    base = wid * _BLOCKS_PER_W
    pltpu.sync_copy(tgt_hbm, tgt_v)

    def do_block(ti, _):
        t = base + ti
        c = t // _SIZE
        pltpu.sync_copy(gidg_hbm.at[t], idxg_v)
        pltpu.sync_copy(gids_hbm.at[t], idxs_v)
        pltpu.sync_copy(w2_hbm.at[pl.ds(t * _ROWS, _ROWS)],
                        out_hbm.at[pl.ds(t * _ROWS, _ROWS)])
        pltpu.async_copy(w2_hbm.at[idxg_v], sel_v, sem).wait()
        for h in range(2):
            pltpu.sync_copy(lgt_hbm.at[c, pl.ds(h * 64, 64)], lg_v)
            for g in range(4):
                b0 = h * 64 + g * 16
                rows_sel = b0 + iota16
                rows_lg = g * 16 + iota16

                def dot_step(i, acc):
                    base = i * 16
                    for u in range(16):
                        col = jnp.full((16,), base + u, jnp.int32)
                        a = plsc.load_gather(sel_v, [rows_sel, col])
                        b = plsc.load_gather(lg_v, [rows_lg, col])
                        acc = acc + a * b
                    return acc

                tot = lax.fori_loop(0, _IN // 16, dot_step,
                                    jnp.zeros((16,), f32))
                outc = jnp.clip(tot, lo, hi)
                sig = 1.0 / (1.0 + jnp.exp(-outc))
                diff = sig - tgt_v[c, pl.ds(b0, 16)]
                olg_v[pl.ds(b0, 16)] = outc
                diff_v[pl.ds(b0, 16)] = diff

            def upd_row(b, _):
                row = h * 64 + b
                dvec = plsc.load_gather(diff_v, [jnp.full((16,), row,
                                                          jnp.int32)])

                for k in range(_IN // 16):
                    cols = pl.ds(k * 16, 16)
                    sc = sel_v[row, cols]
                    lg = lg_v[b, cols]
                    nv = jnp.clip(sc - _LR * dvec * lg,
                                  -_WEIGHT_CLIP, _WEIGHT_CLIP)
                    sel_v[row, cols] = nv
                return 0

            lax.fori_loop(0, 64, upd_row, 0)
        pltpu.async_copy(sel_v, out_hbm.at[idxs_v], sem).wait()
        pltpu.sync_copy(olg_v, olg_hbm.at[t])
        return 0

    lax.fori_loop(0, _BLOCKS_PER_W, do_block, 0)


@jax.jit
def kernel(logits, context, target, context_maps, context_bias, weights):
    f32 = jnp.float32
    bias = context_bias.reshape(_CS, _CMS, 1)

    gidg, gids = pl.pallas_call(
        _index_body,
        grid=(_CLASSES, _SIZE),
        in_specs=[
            pl.BlockSpec((1, 1, _CMS, _CTX), lambda c, s: (c, s, 0, 0)),
            pl.BlockSpec((1, _CMS, 1), lambda c, s: (c * _SIZE + s, 0, 0)),
            pl.BlockSpec((_CTX, _B), lambda c, s: (0, 0)),
        ],
        out_specs=[
            pl.BlockSpec((1, 1, _B), lambda c, s: (c * _SIZE + s, 0, 0)),
            pl.BlockSpec((1, 1, _B), lambda c, s: (c * _SIZE + s, 0, 0)),
        ],
        out_shape=[
            jax.ShapeDtypeStruct((_CS, 1, _B), jnp.int32),
            jax.ShapeDtypeStruct((_CS, 1, _B), jnp.int32),
        ],
    )(context_maps, bias, context.T)
    gidg = gidg.reshape(_CS, _B)
    gids = gids.reshape(_CS, _B)

    w2 = jnp.concatenate(
        [weights.reshape(_CS * _ROWS, _IN),
         jnp.zeros((_NDUMMY, _IN), f32)], axis=0)
    lgt = jnp.transpose(logits, (0, 2, 1))

    mesh = plsc.VectorSubcoreMesh(core_axis_name="c", subcore_axis_name="s")
    sc = pl.kernel(
        _sc_body,
        out_type=[
            jax.ShapeDtypeStruct((_CS * _ROWS + _NDUMMY, _IN), f32),
            jax.ShapeDtypeStruct((_CS, _B), f32),
        ],
        mesh=mesh,
        compiler_params=pltpu.CompilerParams(needs_layout_passes=False),
        scratch_types=[
            pltpu.VMEM((_B, _IN), f32),
            pltpu.VMEM((64, _IN), f32),
            pltpu.VMEM((_B,), jnp.int32),
            pltpu.VMEM((_B,), jnp.int32),
            pltpu.VMEM((_CLASSES, _B), f32),
            pltpu.VMEM((_B,), f32),
            pltpu.VMEM((_B,), f32),
            pltpu.SemaphoreType.DMA,
        ],
    )
    w2_out, olg = sc(w2, gidg, gids, lgt, target)
    new_weights = w2_out[:_CS * _ROWS].reshape(
        _CLASSES, _SIZE, _ROWS, _IN)
    return olg.reshape(_CLASSES, _SIZE, _B), new_weights


# speedup vs baseline: 1.0047x; 1.0047x over previous
import functools
import math

import numpy as _np

import jax
import jax.numpy as jnp
from jax import lax
from jax.experimental import pallas as pl
from jax.experimental.pallas import tpu as pltpu
from jax.experimental.pallas import tpu_sc as plsc

_CLASSES = 10
_SIZE = 64
_CMS = 8
_CTX = 128
_IN = 512
_B = 128
_ROWS = 2 ** _CMS
_LR = 0.01
_PRED_CLIP = 0.01
_WEIGHT_CLIP = 5.0
_CS = _CLASSES * _SIZE
_NDUMMY = 256
_NW = 32
_BLOCKS_PER_W = _CS // _NW


def _logit(x):
    return jnp.log(x) - jnp.log1p(-x)


def _index_body(maps_ref, bias_ref, ctxT_ref, gidg_ref, gids_ref):
    f32 = jnp.float32
    c = pl.program_id(0)
    s = pl.program_id(1)
    t = c * _SIZE + s
    d = jnp.dot(maps_ref[0, 0], ctxT_ref[...],
                preferred_element_type=f32)
    bits = (d > bias_ref[0]).astype(jnp.int32)
    pow2 = jnp.left_shift(
        1, lax.broadcasted_iota(jnp.int32, (_CMS, _B), 0))
    idx = jnp.sum(bits * pow2, axis=0, keepdims=True)
    idx_t = jnp.transpose(idx)
    eq = (idx_t == idx)
    sub_i = lax.broadcasted_iota(jnp.int32, (_B, _B), 0)
    lane_i = lax.broadcasted_iota(jnp.int32, (_B, _B), 1)
    loser = jnp.any(eq & (sub_i > lane_i), axis=0, keepdims=True)
    gidg = t * _ROWS + idx
    lane = lax.broadcasted_iota(jnp.int32, (1, _B), 1)
    dummy = _CS * _ROWS + ((t * _B + lane) % _NDUMMY)
    gidg_ref[0] = gidg
    gids_ref[0] = jnp.where(loser, dummy, gidg)


_LO = float(
    math.log(float(_np.float32(_PRED_CLIP)))
    - math.log1p(-float(_np.float32(_PRED_CLIP))))
_HI = float(
    math.log(float(_np.float32(1.0 - _PRED_CLIP)))
    - math.log1p(-float(_np.float32(1.0 - _PRED_CLIP))))


def _sc_body(w2_hbm, gidg_hbm, gids_hbm, lgt_hbm, tgt_hbm,
             out_hbm, olg_hbm,
             sel_v, lg_v, idxg_v, idxs_v, tgt_v, diff_v, olg_v, dot_v, sem):
    f32 = jnp.float32
    lo = _np.float32(_LO)
    hi = _np.float32(_HI)
    wid = lax.axis_index("s") * 2 + lax.axis_index("c")
    iota16 = lax.broadcasted_iota(jnp.int32, (16,), 0)
    base = wid * _BLOCKS_PER_W
    pltpu.sync_copy(tgt_hbm, tgt_v)

    def do_block(ti, _):
        t = base + ti
        c = t // _SIZE
        pltpu.sync_copy(gidg_hbm.at[t], idxg_v)
        pltpu.sync_copy(gids_hbm.at[t], idxs_v)
        pltpu.sync_copy(w2_hbm.at[pl.ds(t * _ROWS, _ROWS)],
                        out_hbm.at[pl.ds(t * _ROWS, _ROWS)])
        pltpu.async_copy(w2_hbm.at[idxg_v], sel_v, sem).wait()
        for h in range(2):
            pltpu.sync_copy(lgt_hbm.at[c, pl.ds(h * 64, 64)], lg_v)

            def zero_chunk(g, _):
                dot_v[pl.ds(h * 64 + g * 16, 16)] = jnp.zeros((16,), f32)
                return 0

            lax.fori_loop(0, 4, zero_chunk, 0)

            def dot_row(b, _):
                row = h * 64 + b
                acc = jnp.zeros((16,), f32)
                for k in range(_IN // 16):
                    cols = pl.ds(k * 16, 16)
                    acc = acc + sel_v[row, cols] * lg_v[b, cols]
                plsc.addupdate_scatter(
                    dot_v, [jnp.full((16,), row, jnp.int32)], acc)
                return 0

            lax.fori_loop(0, 64, dot_row, 0)

        def vec_chunk(g, _):
            sl = pl.ds(g * 16, 16)
            outc = jnp.clip(dot_v[sl], lo, hi)
            sig = 1.0 / (1.0 + jnp.exp(-outc))
            olg_v[sl] = outc
            diff_v[sl] = sig - tgt_v[c, sl]
            return 0

        lax.fori_loop(0, 8, vec_chunk, 0)

        for h in range(2):
            pltpu.sync_copy(lgt_hbm.at[c, pl.ds(h * 64, 64)], lg_v)

            def upd_row(b, _):
                row = h * 64 + b
                dvec = plsc.load_gather(diff_v, [jnp.full((16,), row,
                                                          jnp.int32)])

                for k in range(_IN // 16):
                    cols = pl.ds(k * 16, 16)
                    sc = sel_v[row, cols]
                    lg = lg_v[b, cols]
                    nv = jnp.clip(sc - _LR * dvec * lg,
                                  -_WEIGHT_CLIP, _WEIGHT_CLIP)
                    sel_v[row, cols] = nv
                return 0

            lax.fori_loop(0, 64, upd_row, 0)
        pltpu.async_copy(sel_v, out_hbm.at[idxs_v], sem).wait()
        pltpu.sync_copy(olg_v, olg_hbm.at[t])
        return 0

    lax.fori_loop(0, _BLOCKS_PER_W, do_block, 0)


@jax.jit
def kernel(logits, context, target, context_maps, context_bias, weights):
    f32 = jnp.float32
    bias = context_bias.reshape(_CS, _CMS, 1)

    gidg, gids = pl.pallas_call(
        _index_body,
        grid=(_CLASSES, _SIZE),
        in_specs=[
            pl.BlockSpec((1, 1, _CMS, _CTX), lambda c, s: (c, s, 0, 0)),
            pl.BlockSpec((1, _CMS, 1), lambda c, s: (c * _SIZE + s, 0, 0)),
            pl.BlockSpec((_CTX, _B), lambda c, s: (0, 0)),
        ],
        out_specs=[
            pl.BlockSpec((1, 1, _B), lambda c, s: (c * _SIZE + s, 0, 0)),
            pl.BlockSpec((1, 1, _B), lambda c, s: (c * _SIZE + s, 0, 0)),
        ],
        out_shape=[
            jax.ShapeDtypeStruct((_CS, 1, _B), jnp.int32),
            jax.ShapeDtypeStruct((_CS, 1, _B), jnp.int32),
        ],
    )(context_maps, bias, context.T)
    gidg = gidg.reshape(_CS, _B)
    gids = gids.reshape(_CS, _B)

    w2 = jnp.concatenate(
        [weights.reshape(_CS * _ROWS, _IN),
         jnp.zeros((_NDUMMY, _IN), f32)], axis=0)
    lgt = jnp.transpose(logits, (0, 2, 1))

    mesh = plsc.VectorSubcoreMesh(core_axis_name="c", subcore_axis_name="s")
    sc = pl.kernel(
        _sc_body,
        out_type=[
            jax.ShapeDtypeStruct((_CS * _ROWS + _NDUMMY, _IN), f32),
            jax.ShapeDtypeStruct((_CS, _B), f32),
        ],
        mesh=mesh,
        compiler_params=pltpu.CompilerParams(needs_layout_passes=False),
        scratch_types=[
            pltpu.VMEM((_B, _IN), f32),
            pltpu.VMEM((64, _IN), f32),
            pltpu.VMEM((_B,), jnp.int32),
            pltpu.VMEM((_B,), jnp.int32),
            pltpu.VMEM((_CLASSES, _B), f32),
            pltpu.VMEM((_B,), f32),
            pltpu.VMEM((_B,), f32),
            pltpu.VMEM((_B,), f32),
            pltpu.SemaphoreType.DMA,
        ],
    )
    w2_out, olg = sc(w2, gidg, gids, lgt, target)
    new_weights = w2_out[:_CS * _ROWS].reshape(
        _CLASSES, _SIZE, _ROWS, _IN)
    return olg.reshape(_CLASSES, _SIZE, _B), new_weights


# speedup vs baseline: 36.7546x; 36.5822x over previous
import jax
import jax.numpy as jnp
from jax.experimental import pallas as pl

_CLASSES = 10
_SIZE = 64
_CMS = 8
_CTX = 128
_IN = 512
_B = 128
_ROWS = 2 ** _CMS
_LR = 0.01
_PRED_CLIP = 0.01
_WEIGHT_CLIP = 5.0
_CS = _CLASSES * _SIZE
_SB = 8


def _logit(x):
    return jnp.log(x) - jnp.log1p(-x)


def _body(maps_ref, bias_ref, ctxT_ref, lg_ref, lgTh_ref, lgTl_ref, tgt_ref,
          w_ref, out_ref, nw_ref):
    f32 = jnp.float32
    lo = _logit(f32(_PRED_CLIP))
    hi = _logit(f32(1.0 - _PRED_CLIP))
    ctxT = ctxT_ref[...]
    lg = lg_ref[0]
    lgT_hi = lgTh_ref[0]
    lgT_lo = lgTl_ref[0]
    tgt = tgt_ref[0]
    for si in range(_SB):
        d = jnp.dot(maps_ref[0, si], ctxT,
                    preferred_element_type=f32)
        bits = (d > bias_ref[0, si]).astype(jnp.int32)
        pow2 = jnp.left_shift(
            1, jax.lax.broadcasted_iota(jnp.int32, (_CMS, _B), 0))
        idx = jnp.sum(bits * pow2, axis=0, keepdims=True)

        w = w_ref[0, si]
        p = jnp.dot(w, lg, preferred_element_type=f32)

        jiota = jax.lax.broadcasted_iota(jnp.int32, (_ROWS, _B), 0)
        onehot_t = jiota == idx
        out = jnp.sum(jnp.where(onehot_t, p, 0.0), axis=0,
                      keepdims=True)
        outc = jnp.clip(out, lo, hi)
        out_ref[0, si] = outc
        diff = jax.nn.sigmoid(outc) - tgt

        biota = jax.lax.broadcasted_iota(jnp.int32, (_ROWS, _B), 1)
        wins = jnp.max(jnp.where(onehot_t, biota, -1),
                       axis=1, keepdims=True)
        ld = _LR * diff
        ld_hi = ld.astype(jnp.bfloat16)
        ld_lo = (ld - ld_hi.astype(f32)).astype(jnp.bfloat16)
        winner = biota == wins
        wsel_hi = jnp.where(winner, ld_hi.astype(f32),
                            0.0).astype(jnp.bfloat16)
        wsel_lo = jnp.where(winner, ld_lo.astype(f32),
                            0.0).astype(jnp.bfloat16)
        upd = (jnp.dot(wsel_hi, lgT_hi, preferred_element_type=f32)
               + jnp.dot(wsel_hi, lgT_lo, preferred_element_type=f32)
               + jnp.dot(wsel_lo, lgT_hi, preferred_element_type=f32))
        nw_ref[0, si] = jnp.where(
            wins >= 0, jnp.clip(w - upd, -_WEIGHT_CLIP, _WEIGHT_CLIP), w)


@jax.jit
def kernel(logits, context, target, context_maps, context_bias, weights):
    f32 = jnp.float32
    ctxT = context.T
    bias = context_bias.reshape(_CLASSES, _SIZE, _CMS, 1)
    lgT = jnp.transpose(logits, (0, 2, 1))
    lgT_hi = lgT.astype(jnp.bfloat16)
    lgT_lo = (lgT - lgT_hi.astype(f32)).astype(jnp.bfloat16)
    tgt = target.reshape(_CLASSES, 1, _B)

    out_logits, new_weights = pl.pallas_call(
        _body,
        grid=(_CLASSES, _SIZE // _SB),
        in_specs=[
            pl.BlockSpec((1, _SB, _CMS, _CTX), lambda c, s: (c, s, 0, 0)),
            pl.BlockSpec((1, _SB, _CMS, 1), lambda c, s: (c, s, 0, 0)),
            pl.BlockSpec((_CTX, _B), lambda c, s: (0, 0)),
            pl.BlockSpec((1, _IN, _B), lambda c, s: (c, 0, 0)),
            pl.BlockSpec((1, _B, _IN), lambda c, s: (c, 0, 0)),
            pl.BlockSpec((1, _B, _IN), lambda c, s: (c, 0, 0)),
            pl.BlockSpec((1, 1, _B), lambda c, s: (c, 0, 0)),
            pl.BlockSpec((1, _SB, _ROWS, _IN), lambda c, s: (c, s, 0, 0)),
        ],
        out_specs=[
            pl.BlockSpec((1, _SB, 1, _B), lambda c, s: (c, s, 0, 0)),
            pl.BlockSpec((1, _SB, _ROWS, _IN), lambda c, s: (c, s, 0, 0)),
        ],
        out_shape=[
            jax.ShapeDtypeStruct((_CLASSES, _SIZE, 1, _B), f32),
            jax.ShapeDtypeStruct((_CLASSES, _SIZE, _ROWS, _IN), f32),
        ],
    )(context_maps, bias, ctxT, logits, lgT_hi, lgT_lo, tgt, weights)
    return out_logits.reshape(_CLASSES, _SIZE, _B), new_weights
